# baseline (speedup 1.0000x reference)
import functools

import jax
import jax.numpy as jnp
from jax import lax
from jax.experimental import pallas as pl
from jax.experimental.pallas import tpu as pltpu
from jax.experimental.pallas import tpu_sc as plsc

_P = 0.12
_NC = 2
_NS = 16
_NW = _NC * _NS
_CHUNK = 8
_NBUF = 4


def _gather_rows(b, s):
    probs = jnp.array([_P / 2, 1.0 - _P, _P / 2], dtype=jnp.float32)
    logits = jnp.log(probs)
    k = jax.random.fold_in(jax.random.key(42), 1)
    off = jax.random.categorical(k, logits, shape=(b, s)) - 1
    off = off.at[:, 0].set(jnp.clip(off[:, 0], 0, 1))
    off = off.at[:, -1].set(jnp.clip(off[:, -1], -1, 0))
    rows = off + jnp.arange(s, dtype=off.dtype)[None, :]
    rows = rows + (jnp.arange(b, dtype=off.dtype) * s)[:, None]
    return rows.reshape(-1).astype(jnp.int32)


@functools.partial(jax.jit, static_argnums=(2, 3))
def _sc_gather(xf, idx, r, c):
    rows_per_w = r // _NW
    nsteps = rows_per_w // _CHUNK
    ngroups = nsteps // _NBUF
    mesh = plsc.VectorSubcoreMesh(core_axis_name="c", subcore_axis_name="s")

    @functools.partial(
        pl.kernel,
        mesh=mesh,
        out_type=jax.ShapeDtypeStruct((r, c), jnp.float32),
        scratch_types=[
            pltpu.VMEM((nsteps, _CHUNK), jnp.int32),
            [pltpu.VMEM((_CHUNK, c), jnp.float32) for _ in range(_NBUF)],
            [pltpu.SemaphoreType.DMA for _ in range(_NBUF)],
            [pltpu.SemaphoreType.DMA for _ in range(_NBUF)],
        ],
    )
    def k(x_hbm, idx_hbm, out_hbm, idx_v, bufs, gsem, wsem):
        wid = lax.axis_index("s") * _NC + lax.axis_index("c")
        base = wid * rows_per_w
        pltpu.sync_copy(idx_hbm.at[wid], idx_v)

        def gather(j, b):
            return pltpu.make_async_copy(x_hbm.at[idx_v.at[j]], bufs[b], gsem[b])

        def write(j, b):
            return pltpu.make_async_copy(
                bufs[b], out_hbm.at[pl.ds(base + j * _CHUNK, _CHUNK)], wsem[b]
            )

        for b in range(_NBUF - 1):
            gather(b, b).start()

        def body(g, _):
            for b in range(_NBUF):
                j = g * _NBUF + b
                gather(j, b).wait()
                write(j, b).start()
                fb = (b + _NBUF - 1) % _NBUF
                if b == 0:
                    @pl.when(g == 0)
                    def _():
                        gather(_NBUF - 1, fb).start()

                    @pl.when(g > 0)
                    def _():
                        write(j - 1, fb).wait()
                        gather(j + _NBUF - 1, fb).start()
                else:
                    @pl.when(g < ngroups - 1)
                    def _():
                        write(j - 1, fb).wait()
                        gather(j + _NBUF - 1, fb).start()
            return 0

        lax.fori_loop(0, ngroups, body, 0)
        for b in range(_NBUF):
            write(nsteps - _NBUF + b, b).wait()

    return k(xf, idx)


def kernel(x):
    b, s, c = x.shape
    r = b * s
    rows = _gather_rows(b, s).reshape(_NW, r // _NW // _CHUNK, _CHUNK)
    out = _sc_gather(x.reshape(r, c), rows, r, c)
    return out.reshape(b, s, c)
